# baseline (speedup 1.0000x reference)
import functools

import jax
import jax.numpy as jnp
from jax import lax
from jax.experimental import pallas as pl
from jax.experimental.pallas import tpu as pltpu
from jax.experimental.pallas import tpu_sc as plsc

MOM = 0.5
TEMP = 0.07
B, D, N, NEG = 256, 128, 1000000, 2048
NW = 32
BPW = B // NW
CHUNK = 128
SCHUNK = 256
NC2 = NEG // SCHUNK
LANES = 16
KSEG = D // LANES


def _sc_dots_body(feat_hbm, bank_hbm, idx_hbm, negidx_hbm,
                  dots_hbm, posf_hbm,
                  feat_v, pidx_v, posr_v, nidx_v, rows0_v, rows1_v, dots_v,
                  sem):
  cid = lax.axis_index("c")
  sid = lax.axis_index("s")
  wid = sid * 2 + cid
  base_b = wid * BPW

  pltpu.sync_copy(feat_hbm.at[pl.ds(base_b, BPW)], feat_v)
  pltpu.sync_copy(idx_hbm.at[pl.ds(base_b, BPW)], pidx_v)
  pltpu.async_copy(bank_hbm.at[pidx_v], posr_v, sem).wait()
  pltpu.sync_copy(posr_v, posf_hbm.at[pl.ds(base_b, BPW)])

  iota16 = lax.iota(jnp.int32, LANES)

  def fire(ci, rbuf):
    i0 = ci * SCHUNK
    pltpu.async_copy(
        bank_hbm.at[nidx_v.at[pl.ds(i0, CHUNK)]], rbuf.at[pl.ds(0, CHUNK)],
        sem)
    pltpu.async_copy(
        bank_hbm.at[nidx_v.at[pl.ds(i0 + CHUNK, CHUNK)]],
        rbuf.at[pl.ds(CHUNK, CHUNK)], sem)

  def drain(rbuf):
    pltpu.make_async_copy(bank_hbm.at[pl.ds(0, SCHUNK)], rbuf, sem).wait()

  def per_feature(bl, carry):
    b = base_b + bl
    pltpu.sync_copy(negidx_hbm.at[pl.ds(b * NEG, NEG)], nidx_v)
    fvecs = [feat_v[bl, pl.ds(k * LANES, LANES)] for k in range(KSEG)]

    def compute(ci, rbuf):
      def per_group(g, carry3):
        base = g * LANES
        res = jnp.zeros((LANES,), jnp.float32)
        for r in range(LANES):
          prods = [rbuf[base + r, pl.ds(k * LANES, LANES)] * fvecs[k]
                   for k in range(KSEG)]
          while len(prods) > 1:
            prods = [x + y for x, y in zip(prods[0::2], prods[1::2])]
          res = jnp.where(iota16 == r, jnp.sum(prods[0]), res)
        dots_v[pl.ds(ci * SCHUNK + g * LANES, LANES)] = res
        return carry3

      lax.fori_loop(0, SCHUNK // LANES, per_group, 0)

    fire(0, rows0_v)

    def per_pair(j, carry2):
      ci0 = 2 * j
      fire(ci0 + 1, rows1_v)
      drain(rows0_v)
      compute(ci0, rows0_v)

      @pl.when(ci0 + 2 < NC2)
      def _():
        fire(ci0 + 2, rows0_v)

      drain(rows1_v)
      compute(ci0 + 1, rows1_v)
      return carry2

    lax.fori_loop(0, NC2 // 2, per_pair, 0)
    pltpu.sync_copy(dots_v, dots_hbm.at[b])
    return carry

  lax.fori_loop(0, BPW, per_feature, 0)


@functools.partial(jax.jit, static_argnames=())
def _sc_dots(feature, bank, idx, neg_idx):
  mesh = plsc.VectorSubcoreMesh(core_axis_name="c", subcore_axis_name="s")
  f = pl.kernel(
      _sc_dots_body,
      out_type=(
          jax.ShapeDtypeStruct((B, NEG), jnp.float32),
          jax.ShapeDtypeStruct((B, D), jnp.float32),
      ),
      mesh=mesh,
      compiler_params=pltpu.CompilerParams(needs_layout_passes=False),
      scratch_types=[
          pltpu.VMEM((BPW, D), jnp.float32),
          pltpu.VMEM((BPW,), jnp.int32),
          pltpu.VMEM((BPW, D), jnp.float32),
          pltpu.VMEM((NEG,), jnp.int32),
          pltpu.VMEM((SCHUNK, D), jnp.float32),
          pltpu.VMEM((SCHUNK, D), jnp.float32),
          pltpu.VMEM((NEG,), jnp.float32),
          pltpu.SemaphoreType.DMA,
      ],
  )
  return f(feature, bank, idx, neg_idx)


NCP = 16
ROWS_PER = N // NCP


def _tc_copy_body(bank_any, out_any, sem):
  for i in range(NCP):
    pltpu.make_async_copy(
        bank_any.at[pl.ds(i * ROWS_PER, ROWS_PER)],
        out_any.at[pl.ds(i * ROWS_PER, ROWS_PER)], sem).start()
  for i in range(NCP):
    pltpu.make_async_copy(
        bank_any.at[pl.ds(i * ROWS_PER, ROWS_PER)],
        out_any.at[pl.ds(i * ROWS_PER, ROWS_PER)], sem).wait()


def _tc_copy(bank):
  return pl.pallas_call(
      _tc_copy_body,
      out_shape=jax.ShapeDtypeStruct((N, D), jnp.float32),
      in_specs=[pl.BlockSpec(memory_space=pl.ANY)],
      out_specs=pl.BlockSpec(memory_space=pl.ANY),
      scratch_shapes=[pltpu.SemaphoreType.DMA],
  )(bank)


def _tc_finish_body(feature_ref, dots_ref, posf_ref, idx_smem, bank_any,
                    loss_ref, out_any, featnew_v, sem):
  f = feature_ref[...]
  inv = 1.0 / jnp.maximum(
      jnp.sqrt(jnp.sum(f * f, axis=1, keepdims=True)), 1e-12)
  posf = posf_ref[...]
  pos_un = jnp.sum(posf * f, axis=1, keepdims=True)
  scale = inv * (1.0 / TEMP)
  pos_l = pos_un * scale
  neg_l = dots_ref[...] * scale
  m = jnp.maximum(jnp.max(neg_l, axis=1, keepdims=True), pos_l)
  se = jnp.sum(jnp.exp(neg_l - m), axis=1, keepdims=True) + jnp.exp(pos_l - m)
  logp0 = pos_l - (jnp.log(se) + m)
  loss_ref[0, 0] = -jnp.mean(logp0)
  fn = MOM * posf + (1.0 - MOM) * (f * inv)
  fn = fn / jnp.maximum(
      jnp.sqrt(jnp.sum(fn * fn, axis=1, keepdims=True)), 1e-12)
  featnew_v[...] = fn

  def fire(i, c):
    pltpu.make_async_copy(
        featnew_v.at[pl.ds(i, 1)], out_any.at[pl.ds(idx_smem[i], 1)], sem
    ).start()
    return c

  lax.fori_loop(0, B, fire, 0)

  def drain(i, c):
    pltpu.make_async_copy(
        featnew_v.at[pl.ds(0, 1)], out_any.at[pl.ds(0, 1)], sem
    ).wait()
    return c

  lax.fori_loop(0, B, drain, 0)


def _tc_finish(feature, dots, posf, idx, bank):
  return pl.pallas_call(
      _tc_finish_body,
      out_shape=(
          jax.ShapeDtypeStruct((1, 1), jnp.float32),
          jax.ShapeDtypeStruct((N, D), jnp.float32),
      ),
      in_specs=[
          pl.BlockSpec(memory_space=pltpu.VMEM),
          pl.BlockSpec(memory_space=pltpu.VMEM),
          pl.BlockSpec(memory_space=pltpu.VMEM),
          pl.BlockSpec(memory_space=pltpu.SMEM),
          pl.BlockSpec(memory_space=pl.ANY),
      ],
      out_specs=(
          pl.BlockSpec(memory_space=pltpu.SMEM),
          pl.BlockSpec(memory_space=pl.ANY),
      ),
      scratch_shapes=[
          pltpu.VMEM((B, D), jnp.float32),
          pltpu.SemaphoreType.DMA,
      ],
      input_output_aliases={4: 1},
  )(feature, dots, posf, idx, bank)


def kernel(feature, bank, idx, neg_idx):
  idx = idx.astype(jnp.int32)
  neg_idx = neg_idx.astype(jnp.int32)
  dots, posf = _sc_dots(feature, bank, idx, neg_idx)
  bank2 = _tc_copy(bank)
  loss_arr, new_bank = _tc_finish(feature, dots, posf, idx, bank2)
  return loss_arr[0, 0], new_bank


# speedup vs baseline: 35.9793x; 35.9793x over previous
import functools

import jax
import jax.numpy as jnp
from jax import lax
from jax.experimental import pallas as pl
from jax.experimental.pallas import tpu as pltpu
from jax.experimental.pallas import tpu_sc as plsc

MOM = 0.5
TEMP = 0.07
B, D, N, NEG = 256, 128, 1000000, 2048
NW = 32
BPW = B // NW
CHUNK = 128
SCHUNK = 256
NC2 = NEG // SCHUNK
LANES = 16
KSEG = D // LANES


def _sc_dots_body(feat_hbm, bank_hbm, idx_hbm, negidx_hbm,
                  dots_hbm, posf_hbm,
                  feat_v, pidx_v, posr_v, nidx_v, rows0_v, rows1_v, dots_v,
                  sem):
  cid = lax.axis_index("c")
  sid = lax.axis_index("s")
  wid = sid * 2 + cid
  base_b = wid * BPW

  pltpu.sync_copy(feat_hbm.at[pl.ds(base_b, BPW)], feat_v)
  pltpu.sync_copy(idx_hbm.at[pl.ds(base_b, BPW)], pidx_v)
  pltpu.async_copy(bank_hbm.at[pidx_v], posr_v, sem).wait()
  pltpu.sync_copy(posr_v, posf_hbm.at[pl.ds(base_b, BPW)])

  iota16 = lax.iota(jnp.int32, LANES)

  def fire(ci, rbuf):
    i0 = ci * SCHUNK
    pltpu.async_copy(
        bank_hbm.at[nidx_v.at[pl.ds(i0, CHUNK)]], rbuf.at[pl.ds(0, CHUNK)],
        sem)
    pltpu.async_copy(
        bank_hbm.at[nidx_v.at[pl.ds(i0 + CHUNK, CHUNK)]],
        rbuf.at[pl.ds(CHUNK, CHUNK)], sem)

  def drain(rbuf):
    pltpu.make_async_copy(bank_hbm.at[pl.ds(0, SCHUNK)], rbuf, sem).wait()

  def per_feature(bl, carry):
    b = base_b + bl
    pltpu.sync_copy(negidx_hbm.at[pl.ds(b * NEG, NEG)], nidx_v)
    fvecs = [feat_v[bl, pl.ds(k * LANES, LANES)] for k in range(KSEG)]

    def compute(ci, rbuf):
      def per_group(g, carry3):
        base = g * LANES
        res = jnp.zeros((LANES,), jnp.float32)
        for r in range(LANES):
          prods = [rbuf[base + r, pl.ds(k * LANES, LANES)] * fvecs[k]
                   for k in range(KSEG)]
          while len(prods) > 1:
            prods = [x + y for x, y in zip(prods[0::2], prods[1::2])]
          res = jnp.where(iota16 == r, jnp.sum(prods[0]), res)
        dots_v[pl.ds(ci * SCHUNK + g * LANES, LANES)] = res
        return carry3

      lax.fori_loop(0, SCHUNK // LANES, per_group, 0)

    fire(0, rows0_v)

    def per_pair(j, carry2):
      ci0 = 2 * j
      fire(ci0 + 1, rows1_v)
      drain(rows0_v)
      compute(ci0, rows0_v)

      @pl.when(ci0 + 2 < NC2)
      def _():
        fire(ci0 + 2, rows0_v)

      drain(rows1_v)
      compute(ci0 + 1, rows1_v)
      return carry2

    lax.fori_loop(0, NC2 // 2, per_pair, 0)
    pltpu.sync_copy(dots_v, dots_hbm.at[b])
    return carry

  lax.fori_loop(0, BPW, per_feature, 0)


@functools.partial(jax.jit, static_argnames=())
def _sc_dots(feature, bank, idx, neg_idx):
  mesh = plsc.VectorSubcoreMesh(core_axis_name="c", subcore_axis_name="s")
  f = pl.kernel(
      _sc_dots_body,
      out_type=(
          jax.ShapeDtypeStruct((B, NEG), jnp.float32),
          jax.ShapeDtypeStruct((B, D), jnp.float32),
      ),
      mesh=mesh,
      compiler_params=pltpu.CompilerParams(needs_layout_passes=False),
      scratch_types=[
          pltpu.VMEM((BPW, D), jnp.float32),
          pltpu.VMEM((BPW,), jnp.int32),
          pltpu.VMEM((BPW, D), jnp.float32),
          pltpu.VMEM((NEG,), jnp.int32),
          pltpu.VMEM((SCHUNK, D), jnp.float32),
          pltpu.VMEM((SCHUNK, D), jnp.float32),
          pltpu.VMEM((NEG,), jnp.float32),
          pltpu.SemaphoreType.DMA,
      ],
  )
  return f(feature, bank, idx, neg_idx)


CPBLK = 8000


def _tc_copy_body(src_ref, dst_ref):
  dst_ref[...] = src_ref[...]


def _tc_copy(bank):
  return pl.pallas_call(
      _tc_copy_body,
      out_shape=jax.ShapeDtypeStruct((N, D), jnp.float32),
      grid=(N // CPBLK,),
      in_specs=[pl.BlockSpec((CPBLK, D), lambda i: (i, 0))],
      out_specs=pl.BlockSpec((CPBLK, D), lambda i: (i, 0)),
  )(bank)


def _tc_finish_body(feature_ref, dots_ref, posf_ref, idx_smem, bank_any,
                    loss_ref, out_any, featnew_v, sem):
  f = feature_ref[...]
  inv = 1.0 / jnp.maximum(
      jnp.sqrt(jnp.sum(f * f, axis=1, keepdims=True)), 1e-12)
  posf = posf_ref[...]
  pos_un = jnp.sum(posf * f, axis=1, keepdims=True)
  scale = inv * (1.0 / TEMP)
  pos_l = pos_un * scale
  neg_l = dots_ref[...] * scale
  m = jnp.maximum(jnp.max(neg_l, axis=1, keepdims=True), pos_l)
  se = jnp.sum(jnp.exp(neg_l - m), axis=1, keepdims=True) + jnp.exp(pos_l - m)
  logp0 = pos_l - (jnp.log(se) + m)
  loss_ref[0, 0] = -jnp.mean(logp0)
  fn = MOM * posf + (1.0 - MOM) * (f * inv)
  fn = fn / jnp.maximum(
      jnp.sqrt(jnp.sum(fn * fn, axis=1, keepdims=True)), 1e-12)
  featnew_v[...] = fn

  def fire(i, c):
    pltpu.make_async_copy(
        featnew_v.at[pl.ds(i, 1)], out_any.at[pl.ds(idx_smem[i], 1)], sem
    ).start()
    return c

  lax.fori_loop(0, B, fire, 0)

  def drain(i, c):
    pltpu.make_async_copy(
        featnew_v.at[pl.ds(0, 1)], out_any.at[pl.ds(0, 1)], sem
    ).wait()
    return c

  lax.fori_loop(0, B, drain, 0)


def _tc_finish(feature, dots, posf, idx, bank):
  return pl.pallas_call(
      _tc_finish_body,
      out_shape=(
          jax.ShapeDtypeStruct((1, 1), jnp.float32),
          jax.ShapeDtypeStruct((N, D), jnp.float32),
      ),
      in_specs=[
          pl.BlockSpec(memory_space=pltpu.VMEM),
          pl.BlockSpec(memory_space=pltpu.VMEM),
          pl.BlockSpec(memory_space=pltpu.VMEM),
          pl.BlockSpec(memory_space=pltpu.SMEM),
          pl.BlockSpec(memory_space=pl.ANY),
      ],
      out_specs=(
          pl.BlockSpec(memory_space=pltpu.SMEM),
          pl.BlockSpec(memory_space=pl.ANY),
      ),
      scratch_shapes=[
          pltpu.VMEM((B, D), jnp.float32),
          pltpu.SemaphoreType.DMA,
      ],
      input_output_aliases={4: 1},
  )(feature, dots, posf, idx, bank)


def kernel(feature, bank, idx, neg_idx):
  idx = idx.astype(jnp.int32)
  neg_idx = neg_idx.astype(jnp.int32)
  dots, posf = _sc_dots(feature, bank, idx, neg_idx)
  bank2 = _tc_copy(bank)
  loss_arr, new_bank = _tc_finish(feature, dots, posf, idx, bank2)
  return loss_arr[0, 0], new_bank


# speedup vs baseline: 36.1024x; 1.0034x over previous
import functools

import jax
import jax.numpy as jnp
from jax import lax
from jax.experimental import pallas as pl
from jax.experimental.pallas import tpu as pltpu
from jax.experimental.pallas import tpu_sc as plsc

MOM = 0.5
TEMP = 0.07
B, D, N, NEG = 256, 128, 1000000, 2048
NW = 32
BPW = B // NW
CHUNK = 128
SCHUNK = 256
NC2 = NEG // SCHUNK
LANES = 16
KSEG = D // LANES


def _sc_dots_body(feat_hbm, bank_hbm, idx_hbm, negidx_hbm,
                  dots_hbm, posf_hbm,
                  feat_v, pidx_v, posr_v, nidx_v, rows0_v, rows1_v, dots_v,
                  sem):
  cid = lax.axis_index("c")
  sid = lax.axis_index("s")
  wid = sid * 2 + cid
  base_b = wid * BPW

  pltpu.sync_copy(feat_hbm.at[pl.ds(base_b, BPW)], feat_v)
  pltpu.sync_copy(idx_hbm.at[pl.ds(base_b, BPW)], pidx_v)
  pltpu.async_copy(bank_hbm.at[pidx_v], posr_v, sem).wait()
  pltpu.sync_copy(posr_v, posf_hbm.at[pl.ds(base_b, BPW)])

  iota16 = lax.iota(jnp.int32, LANES)

  def fire(ci, rbuf):
    i0 = ci * SCHUNK
    pltpu.async_copy(
        bank_hbm.at[nidx_v.at[pl.ds(i0, CHUNK)]], rbuf.at[pl.ds(0, CHUNK)],
        sem)
    pltpu.async_copy(
        bank_hbm.at[nidx_v.at[pl.ds(i0 + CHUNK, CHUNK)]],
        rbuf.at[pl.ds(CHUNK, CHUNK)], sem)

  def drain(rbuf):
    pltpu.make_async_copy(bank_hbm.at[pl.ds(0, SCHUNK)], rbuf, sem).wait()

  def per_feature(bl, carry):
    b = base_b + bl
    pltpu.sync_copy(negidx_hbm.at[pl.ds(b * NEG, NEG)], nidx_v)
    fvecs = [feat_v[bl, pl.ds(k * LANES, LANES)] for k in range(KSEG)]

    def compute(ci, rbuf):
      def per_group(g, carry3):
        base = g * LANES
        res = jnp.zeros((LANES,), jnp.float32)
        for r in range(LANES):
          prods = [rbuf[base + r, pl.ds(k * LANES, LANES)] * fvecs[k]
                   for k in range(KSEG)]
          while len(prods) > 1:
            prods = [x + y for x, y in zip(prods[0::2], prods[1::2])]
          res = jnp.where(iota16 == r, jnp.sum(prods[0]), res)
        dots_v[pl.ds(ci * SCHUNK + g * LANES, LANES)] = res
        return carry3

      lax.fori_loop(0, SCHUNK // LANES, per_group, 0)

    fire(0, rows0_v)

    def per_pair(j, carry2):
      ci0 = 2 * j
      fire(ci0 + 1, rows1_v)
      drain(rows0_v)
      compute(ci0, rows0_v)

      @pl.when(ci0 + 2 < NC2)
      def _():
        fire(ci0 + 2, rows0_v)

      drain(rows1_v)
      compute(ci0 + 1, rows1_v)
      return carry2

    lax.fori_loop(0, NC2 // 2, per_pair, 0)
    pltpu.sync_copy(dots_v, dots_hbm.at[b])
    return carry

  lax.fori_loop(0, BPW, per_feature, 0)


@functools.partial(jax.jit, static_argnames=())
def _sc_dots(feature, bank, idx, neg_idx):
  mesh = plsc.VectorSubcoreMesh(core_axis_name="c", subcore_axis_name="s")
  f = pl.kernel(
      _sc_dots_body,
      out_type=(
          jax.ShapeDtypeStruct((B, NEG), jnp.float32),
          jax.ShapeDtypeStruct((B, D), jnp.float32),
      ),
      mesh=mesh,
      compiler_params=pltpu.CompilerParams(needs_layout_passes=False),
      scratch_types=[
          pltpu.VMEM((BPW, D), jnp.float32),
          pltpu.VMEM((BPW,), jnp.int32),
          pltpu.VMEM((BPW, D), jnp.float32),
          pltpu.VMEM((NEG,), jnp.int32),
          pltpu.VMEM((SCHUNK, D), jnp.float32),
          pltpu.VMEM((SCHUNK, D), jnp.float32),
          pltpu.VMEM((NEG,), jnp.float32),
          pltpu.SemaphoreType.DMA,
      ],
  )
  return f(feature, bank, idx, neg_idx)


CPBLK = 10000


def _tc_copy_body(src_ref, dst_ref):
  dst_ref[...] = src_ref[...]


def _tc_copy(bank):
  return pl.pallas_call(
      _tc_copy_body,
      out_shape=jax.ShapeDtypeStruct((N, D), jnp.float32),
      grid=(N // CPBLK,),
      in_specs=[pl.BlockSpec((CPBLK, D), lambda i: (i, 0))],
      out_specs=pl.BlockSpec((CPBLK, D), lambda i: (i, 0)),
  )(bank)


def _tc_finish_body(feature_ref, dots_ref, posf_ref, idx_smem, bank_any,
                    loss_ref, out_any, featnew_v, sem):
  f = feature_ref[...]
  inv = 1.0 / jnp.maximum(
      jnp.sqrt(jnp.sum(f * f, axis=1, keepdims=True)), 1e-12)
  posf = posf_ref[...]
  pos_un = jnp.sum(posf * f, axis=1, keepdims=True)
  scale = inv * (1.0 / TEMP)
  pos_l = pos_un * scale
  neg_l = dots_ref[...] * scale
  m = jnp.maximum(jnp.max(neg_l, axis=1, keepdims=True), pos_l)
  se = jnp.sum(jnp.exp(neg_l - m), axis=1, keepdims=True) + jnp.exp(pos_l - m)
  logp0 = pos_l - (jnp.log(se) + m)
  loss_ref[0, 0] = -jnp.mean(logp0)
  fn = MOM * posf + (1.0 - MOM) * (f * inv)
  fn = fn / jnp.maximum(
      jnp.sqrt(jnp.sum(fn * fn, axis=1, keepdims=True)), 1e-12)
  featnew_v[...] = fn

  def fire(i, c):
    pltpu.make_async_copy(
        featnew_v.at[pl.ds(i, 1)], out_any.at[pl.ds(idx_smem[i], 1)], sem
    ).start()
    return c

  lax.fori_loop(0, B, fire, 0)

  def drain(i, c):
    pltpu.make_async_copy(
        featnew_v.at[pl.ds(0, 1)], out_any.at[pl.ds(0, 1)], sem
    ).wait()
    return c

  lax.fori_loop(0, B, drain, 0)


def _tc_finish(feature, dots, posf, idx, bank):
  return pl.pallas_call(
      _tc_finish_body,
      out_shape=(
          jax.ShapeDtypeStruct((1, 1), jnp.float32),
          jax.ShapeDtypeStruct((N, D), jnp.float32),
      ),
      in_specs=[
          pl.BlockSpec(memory_space=pltpu.VMEM),
          pl.BlockSpec(memory_space=pltpu.VMEM),
          pl.BlockSpec(memory_space=pltpu.VMEM),
          pl.BlockSpec(memory_space=pltpu.SMEM),
          pl.BlockSpec(memory_space=pl.ANY),
      ],
      out_specs=(
          pl.BlockSpec(memory_space=pltpu.SMEM),
          pl.BlockSpec(memory_space=pl.ANY),
      ),
      scratch_shapes=[
          pltpu.VMEM((B, D), jnp.float32),
          pltpu.SemaphoreType.DMA,
      ],
      input_output_aliases={4: 1},
  )(feature, dots, posf, idx, bank)


def kernel(feature, bank, idx, neg_idx):
  idx = idx.astype(jnp.int32)
  neg_idx = neg_idx.astype(jnp.int32)
  dots, posf = _sc_dots(feature, bank, idx, neg_idx)
  bank2 = _tc_copy(bank)
  loss_arr, new_bank = _tc_finish(feature, dots, posf, idx, bank2)
  return loss_arr[0, 0], new_bank


# speedup vs baseline: 36.2260x; 1.0034x over previous
import functools

import jax
import jax.numpy as jnp
from jax import lax
from jax.experimental import pallas as pl
from jax.experimental.pallas import tpu as pltpu
from jax.experimental.pallas import tpu_sc as plsc

MOM = 0.5
TEMP = 0.07
B, D, N, NEG = 256, 128, 1000000, 2048
NW = 32
BPW = B // NW
CHUNK = 128
SCHUNK = 256
NC2 = NEG // SCHUNK
LANES = 16
KSEG = D // LANES


def _sc_dots_body(feat_hbm, bank_hbm, idx_hbm, negidx_hbm,
                  dots_hbm, posf_hbm,
                  feat_v, pidx_v, posr_v, nidx_v, rows0_v, rows1_v, dots_v,
                  sem):
  cid = lax.axis_index("c")
  sid = lax.axis_index("s")
  wid = sid * 2 + cid
  base_b = wid * BPW

  pltpu.sync_copy(feat_hbm.at[pl.ds(base_b, BPW)], feat_v)
  pltpu.sync_copy(idx_hbm.at[pl.ds(base_b, BPW)], pidx_v)
  pltpu.async_copy(bank_hbm.at[pidx_v], posr_v, sem).wait()
  pltpu.sync_copy(posr_v, posf_hbm.at[pl.ds(base_b, BPW)])

  iota16 = lax.iota(jnp.int32, LANES)

  def fire(ci, rbuf):
    i0 = ci * SCHUNK
    pltpu.async_copy(
        bank_hbm.at[nidx_v.at[pl.ds(i0, CHUNK)]], rbuf.at[pl.ds(0, CHUNK)],
        sem)
    pltpu.async_copy(
        bank_hbm.at[nidx_v.at[pl.ds(i0 + CHUNK, CHUNK)]],
        rbuf.at[pl.ds(CHUNK, CHUNK)], sem)

  def drain(rbuf):
    pltpu.make_async_copy(bank_hbm.at[pl.ds(0, SCHUNK)], rbuf, sem).wait()

  def per_feature(bl, carry):
    b = base_b + bl
    pltpu.sync_copy(negidx_hbm.at[pl.ds(b * NEG, NEG)], nidx_v)
    fvecs = [feat_v[bl, pl.ds(k * LANES, LANES)] for k in range(KSEG)]

    def compute(ci, rbuf):
      def per_group(g, carry3):
        base = g * LANES
        res = jnp.zeros((LANES,), jnp.float32)
        for r in range(LANES):
          prods = [rbuf[base + r, pl.ds(k * LANES, LANES)] * fvecs[k]
                   for k in range(KSEG)]
          while len(prods) > 1:
            prods = [x + y for x, y in zip(prods[0::2], prods[1::2])]
          res = jnp.where(iota16 == r, jnp.sum(prods[0]), res)
        dots_v[pl.ds(ci * SCHUNK + g * LANES, LANES)] = res
        return carry3

      lax.fori_loop(0, SCHUNK // LANES, per_group, 0)

    fire(0, rows0_v)

    def per_pair(j, carry2):
      ci0 = 2 * j
      fire(ci0 + 1, rows1_v)
      drain(rows0_v)
      compute(ci0, rows0_v)

      @pl.when(ci0 + 2 < NC2)
      def _():
        fire(ci0 + 2, rows0_v)

      drain(rows1_v)
      compute(ci0 + 1, rows1_v)
      return carry2

    lax.fori_loop(0, NC2 // 2, per_pair, 0)
    pltpu.sync_copy(dots_v, dots_hbm.at[b])
    return carry

  lax.fori_loop(0, BPW, per_feature, 0)


@functools.partial(jax.jit, static_argnames=())
def _sc_dots(feature, bank, idx, neg_idx):
  mesh = plsc.VectorSubcoreMesh(core_axis_name="c", subcore_axis_name="s")
  f = pl.kernel(
      _sc_dots_body,
      out_type=(
          jax.ShapeDtypeStruct((B, NEG), jnp.float32),
          jax.ShapeDtypeStruct((B, D), jnp.float32),
      ),
      mesh=mesh,
      compiler_params=pltpu.CompilerParams(needs_layout_passes=False),
      scratch_types=[
          pltpu.VMEM((BPW, D), jnp.float32),
          pltpu.VMEM((BPW,), jnp.int32),
          pltpu.VMEM((BPW, D), jnp.float32),
          pltpu.VMEM((NEG,), jnp.int32),
          pltpu.VMEM((SCHUNK, D), jnp.float32),
          pltpu.VMEM((SCHUNK, D), jnp.float32),
          pltpu.VMEM((NEG,), jnp.float32),
          pltpu.SemaphoreType.DMA,
      ],
  )
  return f(feature, bank, idx, neg_idx)


CPBLK = 20000


def _tc_copy_body(src_ref, dst_ref):
  dst_ref[...] = src_ref[...]


def _tc_copy(bank):
  return pl.pallas_call(
      _tc_copy_body,
      out_shape=jax.ShapeDtypeStruct((N, D), jnp.float32),
      grid=(N // CPBLK,),
      in_specs=[pl.BlockSpec((CPBLK, D), lambda i: (i, 0))],
      out_specs=pl.BlockSpec((CPBLK, D), lambda i: (i, 0)),
  )(bank)


def _tc_finish_body(feature_ref, dots_ref, posf_ref, idx_smem, bank_any,
                    loss_ref, out_any, featnew_v, sem):
  f = feature_ref[...]
  inv = 1.0 / jnp.maximum(
      jnp.sqrt(jnp.sum(f * f, axis=1, keepdims=True)), 1e-12)
  posf = posf_ref[...]
  pos_un = jnp.sum(posf * f, axis=1, keepdims=True)
  scale = inv * (1.0 / TEMP)
  pos_l = pos_un * scale
  neg_l = dots_ref[...] * scale
  m = jnp.maximum(jnp.max(neg_l, axis=1, keepdims=True), pos_l)
  se = jnp.sum(jnp.exp(neg_l - m), axis=1, keepdims=True) + jnp.exp(pos_l - m)
  logp0 = pos_l - (jnp.log(se) + m)
  loss_ref[0, 0] = -jnp.mean(logp0)
  fn = MOM * posf + (1.0 - MOM) * (f * inv)
  fn = fn / jnp.maximum(
      jnp.sqrt(jnp.sum(fn * fn, axis=1, keepdims=True)), 1e-12)
  featnew_v[...] = fn

  def fire(i, c):
    pltpu.make_async_copy(
        featnew_v.at[pl.ds(i, 1)], out_any.at[pl.ds(idx_smem[i], 1)], sem
    ).start()
    return c

  lax.fori_loop(0, B, fire, 0)

  def drain(i, c):
    pltpu.make_async_copy(
        featnew_v.at[pl.ds(0, 1)], out_any.at[pl.ds(0, 1)], sem
    ).wait()
    return c

  lax.fori_loop(0, B, drain, 0)


def _tc_finish(feature, dots, posf, idx, bank):
  return pl.pallas_call(
      _tc_finish_body,
      out_shape=(
          jax.ShapeDtypeStruct((1, 1), jnp.float32),
          jax.ShapeDtypeStruct((N, D), jnp.float32),
      ),
      in_specs=[
          pl.BlockSpec(memory_space=pltpu.VMEM),
          pl.BlockSpec(memory_space=pltpu.VMEM),
          pl.BlockSpec(memory_space=pltpu.VMEM),
          pl.BlockSpec(memory_space=pltpu.SMEM),
          pl.BlockSpec(memory_space=pl.ANY),
      ],
      out_specs=(
          pl.BlockSpec(memory_space=pltpu.SMEM),
          pl.BlockSpec(memory_space=pl.ANY),
      ),
      scratch_shapes=[
          pltpu.VMEM((B, D), jnp.float32),
          pltpu.SemaphoreType.DMA,
      ],
      input_output_aliases={4: 1},
  )(feature, dots, posf, idx, bank)


def kernel(feature, bank, idx, neg_idx):
  idx = idx.astype(jnp.int32)
  neg_idx = neg_idx.astype(jnp.int32)
  dots, posf = _sc_dots(feature, bank, idx, neg_idx)
  bank2 = _tc_copy(bank)
  loss_arr, new_bank = _tc_finish(feature, dots, posf, idx, bank2)
  return loss_arr[0, 0], new_bank


# speedup vs baseline: 36.2369x; 1.0003x over previous
import functools

import jax
import jax.numpy as jnp
from jax import lax
from jax.experimental import pallas as pl
from jax.experimental.pallas import tpu as pltpu
from jax.experimental.pallas import tpu_sc as plsc

MOM = 0.5
TEMP = 0.07
B, D, N, NEG = 256, 128, 1000000, 2048
NW = 32
BPW = B // NW
CHUNK = 128
SCHUNK = 256
NC2 = NEG // SCHUNK
LANES = 16
KSEG = D // LANES


def _sc_dots_body(feat_hbm, bank_hbm, idx_hbm, negidx_hbm,
                  dots_hbm, posf_hbm,
                  feat_v, pidx_v, posr_v, nidx_v, rows0_v, rows1_v, dots_v,
                  sem):
  cid = lax.axis_index("c")
  sid = lax.axis_index("s")
  wid = sid * 2 + cid
  base_b = wid * BPW

  pltpu.sync_copy(feat_hbm.at[pl.ds(base_b, BPW)], feat_v)
  pltpu.sync_copy(idx_hbm.at[pl.ds(base_b, BPW)], pidx_v)
  pltpu.async_copy(bank_hbm.at[pidx_v], posr_v, sem).wait()
  pltpu.sync_copy(posr_v, posf_hbm.at[pl.ds(base_b, BPW)])
  pltpu.sync_copy(negidx_hbm.at[pl.ds(base_b * NEG, BPW * NEG)], nidx_v)

  iota16 = lax.iota(jnp.int32, LANES)
  TCH = BPW * NEG // SCHUNK

  def fire(t, rbuf):
    i0 = t * SCHUNK
    pltpu.async_copy(
        bank_hbm.at[nidx_v.at[pl.ds(i0, CHUNK)]], rbuf.at[pl.ds(0, CHUNK)],
        sem)
    pltpu.async_copy(
        bank_hbm.at[nidx_v.at[pl.ds(i0 + CHUNK, CHUNK)]],
        rbuf.at[pl.ds(CHUNK, CHUNK)], sem)

  def drain(rbuf):
    pltpu.make_async_copy(bank_hbm.at[pl.ds(0, SCHUNK)], rbuf, sem).wait()

  NCPF = NEG // SCHUNK

  def compute(t, rbuf):
    bl = t // NCPF
    off = (t - bl * NCPF) * SCHUNK
    fvecs = [feat_v[bl, pl.ds(k * LANES, LANES)] for k in range(KSEG)]

    def per_group(g, carry3):
      base = g * LANES
      res = jnp.zeros((LANES,), jnp.float32)
      for r in range(LANES):
        prods = [rbuf[base + r, pl.ds(k * LANES, LANES)] * fvecs[k]
                 for k in range(KSEG)]
        while len(prods) > 1:
          prods = [x + y for x, y in zip(prods[0::2], prods[1::2])]
        res = jnp.where(iota16 == r, jnp.sum(prods[0]), res)
      dots_v[bl, pl.ds(off + g * LANES, LANES)] = res
      return carry3

    lax.fori_loop(0, SCHUNK // LANES, per_group, 0)

  fire(0, rows0_v)

  def per_pair(j, carry2):
    t0 = 2 * j
    fire(t0 + 1, rows1_v)
    drain(rows0_v)
    compute(t0, rows0_v)

    @pl.when(t0 + 2 < TCH)
    def _():
      fire(t0 + 2, rows0_v)

    drain(rows1_v)
    compute(t0 + 1, rows1_v)
    return carry2

  lax.fori_loop(0, TCH // 2, per_pair, 0)
  pltpu.sync_copy(dots_v, dots_hbm.at[pl.ds(base_b, BPW)])


@functools.partial(jax.jit, static_argnames=())
def _sc_dots(feature, bank, idx, neg_idx):
  mesh = plsc.VectorSubcoreMesh(core_axis_name="c", subcore_axis_name="s")
  f = pl.kernel(
      _sc_dots_body,
      out_type=(
          jax.ShapeDtypeStruct((B, NEG), jnp.float32),
          jax.ShapeDtypeStruct((B, D), jnp.float32),
      ),
      mesh=mesh,
      compiler_params=pltpu.CompilerParams(needs_layout_passes=False),
      scratch_types=[
          pltpu.VMEM((BPW, D), jnp.float32),
          pltpu.VMEM((BPW,), jnp.int32),
          pltpu.VMEM((BPW, D), jnp.float32),
          pltpu.VMEM((BPW * NEG,), jnp.int32),
          pltpu.VMEM((SCHUNK, D), jnp.float32),
          pltpu.VMEM((SCHUNK, D), jnp.float32),
          pltpu.VMEM((BPW, NEG), jnp.float32),
          pltpu.SemaphoreType.DMA,
      ],
  )
  return f(feature, bank, idx, neg_idx)


CPBLK = 20000


def _tc_copy_body(src_ref, dst_ref):
  dst_ref[...] = src_ref[...]


def _tc_copy(bank):
  return pl.pallas_call(
      _tc_copy_body,
      out_shape=jax.ShapeDtypeStruct((N, D), jnp.float32),
      grid=(N // CPBLK,),
      in_specs=[pl.BlockSpec((CPBLK, D), lambda i: (i, 0))],
      out_specs=pl.BlockSpec((CPBLK, D), lambda i: (i, 0)),
  )(bank)


def _tc_finish_body(feature_ref, dots_ref, posf_ref, idx_smem, bank_any,
                    loss_ref, out_any, featnew_v, sem):
  f = feature_ref[...]
  inv = 1.0 / jnp.maximum(
      jnp.sqrt(jnp.sum(f * f, axis=1, keepdims=True)), 1e-12)
  posf = posf_ref[...]
  pos_un = jnp.sum(posf * f, axis=1, keepdims=True)
  scale = inv * (1.0 / TEMP)
  pos_l = pos_un * scale
  neg_l = dots_ref[...] * scale
  m = jnp.maximum(jnp.max(neg_l, axis=1, keepdims=True), pos_l)
  se = jnp.sum(jnp.exp(neg_l - m), axis=1, keepdims=True) + jnp.exp(pos_l - m)
  logp0 = pos_l - (jnp.log(se) + m)
  loss_ref[0, 0] = -jnp.mean(logp0)
  fn = MOM * posf + (1.0 - MOM) * (f * inv)
  fn = fn / jnp.maximum(
      jnp.sqrt(jnp.sum(fn * fn, axis=1, keepdims=True)), 1e-12)
  featnew_v[...] = fn

  def fire(i, c):
    pltpu.make_async_copy(
        featnew_v.at[pl.ds(i, 1)], out_any.at[pl.ds(idx_smem[i], 1)], sem
    ).start()
    return c

  lax.fori_loop(0, B, fire, 0)

  def drain(i, c):
    pltpu.make_async_copy(
        featnew_v.at[pl.ds(0, 1)], out_any.at[pl.ds(0, 1)], sem
    ).wait()
    return c

  lax.fori_loop(0, B, drain, 0)


def _tc_finish(feature, dots, posf, idx, bank):
  return pl.pallas_call(
      _tc_finish_body,
      out_shape=(
          jax.ShapeDtypeStruct((1, 1), jnp.float32),
          jax.ShapeDtypeStruct((N, D), jnp.float32),
      ),
      in_specs=[
          pl.BlockSpec(memory_space=pltpu.VMEM),
          pl.BlockSpec(memory_space=pltpu.VMEM),
          pl.BlockSpec(memory_space=pltpu.VMEM),
          pl.BlockSpec(memory_space=pltpu.SMEM),
          pl.BlockSpec(memory_space=pl.ANY),
      ],
      out_specs=(
          pl.BlockSpec(memory_space=pltpu.SMEM),
          pl.BlockSpec(memory_space=pl.ANY),
      ),
      scratch_shapes=[
          pltpu.VMEM((B, D), jnp.float32),
          pltpu.SemaphoreType.DMA,
      ],
      input_output_aliases={4: 1},
  )(feature, dots, posf, idx, bank)


def kernel(feature, bank, idx, neg_idx):
  idx = idx.astype(jnp.int32)
  neg_idx = neg_idx.astype(jnp.int32)
  dots, posf = _sc_dots(feature, bank, idx, neg_idx)
  bank2 = _tc_copy(bank)
  loss_arr, new_bank = _tc_finish(feature, dots, posf, idx, bank2)
  return loss_arr[0, 0], new_bank
